# Initial kernel scaffold; baseline (speedup 1.0000x reference)
import jax
import jax.numpy as jnp
from jax.experimental import pallas as pl


def kernel(x, W_off, b_off, W_d, b_d):
    raise NotImplementedError("write your pallas kernel here")


# speedup vs baseline: 1402.7475x; 1402.7475x over previous
import functools

import jax
import jax.numpy as jnp
import numpy as np
from jax import lax
from jax.experimental import pallas as pl
from jax.experimental.pallas import tpu as pltpu
from jax.experimental.pallas import tpu_sc as plsc

KS = 3
PAD = 1
C = 96
NTAP = 9
H = 222
HW = H * H
HP = 226
NROWS = HP * HP

NW = 32
P_TILE = 1568
NPIX = NW * P_TILE
PB = 16
NB = P_TILE // PB
TAPS4 = 4 * NTAP
BLK = PB * TAPS4
GCH = 128
NG = (BLK + GCH - 1) // GCH

_mesh = plsc.VectorSubcoreMesh(core_axis_name="c", subcore_axis_name="s")


@functools.partial(
    pl.kernel,
    mesh=_mesh,
    out_type=jax.ShapeDtypeStruct((NPIX, C), jnp.float32),
    scratch_types=[
        pltpu.VMEM((BLK,), jnp.int32),
        pltpu.VMEM((BLK + 16,), jnp.float32),
        pltpu.VMEM((BLK, 128), jnp.float32),
        pltpu.VMEM((PB, C), jnp.float32),
        pltpu.SemaphoreType.DMA,
    ],
)
def _sc_gather_combine(table_hbm, idx_hbm, w_hbm, out_hbm,
                       idx_v, w_v, rows_v, out_v, sem):
    wid = lax.axis_index("s") * 2 + lax.axis_index("c")
    base_pix = wid * P_TILE

    def block(b, carry):
        off = (base_pix + b * PB) * TAPS4
        pltpu.sync_copy(idx_hbm.at[pl.ds(off, BLK)], idx_v)
        pltpu.sync_copy(w_hbm.at[pl.ds(off, BLK)], w_v.at[pl.ds(0, BLK)])
        handles = []
        for k in range(NG):
            cnt = min(GCH, BLK - k * GCH)
            handles.append(pltpu.async_copy(
                table_hbm.at[idx_v.at[pl.ds(k * GCH, cnt)]],
                rows_v.at[pl.ds(k * GCH, cnt)], sem))
        for h in handles:
            h.wait()

        def pix_body(i, c2):
            base = i * TAPS4
            wvecs = tuple(w_v[pl.ds(base + 16 * g, 16)] for g in range(3))
            acc = [jnp.zeros((16,), jnp.float32) for _ in range(C // 16)]
            for t in range(TAPS4):
                wv = lax.broadcast_in_dim(wvecs[t // 16][t % 16], (16,), ())
                for c in range(C // 16):
                    acc[c] = acc[c] + wv * rows_v[base + t, pl.ds(c * 16, 16)]
            for c in range(C // 16):
                out_v[i, pl.ds(c * 16, 16)] = acc[c]
            return c2
        lax.fori_loop(0, PB, pix_body, 0)
        pltpu.sync_copy(out_v, out_hbm.at[pl.ds(base_pix + b * PB, PB)])
        return carry

    lax.fori_loop(0, NB, block, 0)


def _indices_and_weights(offset):
    ox = offset[0::2]
    oy = offset[1::2]
    dxn = jnp.asarray(np.repeat(np.arange(-1, 2), 3), jnp.float32)
    dyn = jnp.asarray(np.tile(np.arange(-1, 2), 3), jnp.float32)
    grid = jnp.arange(1, H + 1, dtype=jnp.float32)
    px = ox + (dxn[:, None, None] + grid[None, :, None])
    py = oy + (dyn[:, None, None] + grid[None, None, :])

    def axis_terms(p):
        f = jnp.floor(p)
        q0 = jnp.clip(f, 0, HP - 1)
        q1 = jnp.clip(f + 1, 0, HP - 1)
        masked = jnp.logical_or(p < PAD, p > HP - 1 - PAD)
        pu = jnp.clip(jnp.where(masked, f, p), 0, HP - 1)
        w0 = 1.0 + q0 - pu
        w1 = 1.0 - (q1 - pu)
        return q0.astype(jnp.int32), q1.astype(jnp.int32), w0, w1

    qx0, qx1, wx0, wx1 = axis_terms(px)
    qy0, qy1, wy0, wy1 = axis_terms(py)

    nbase = (jnp.arange(NTAP, dtype=jnp.int32) * NROWS)[:, None, None]
    idx = jnp.stack([nbase + qx0 * HP + qy0,
                     nbase + qx1 * HP + qy1,
                     nbase + qx0 * HP + qy1,
                     nbase + qx1 * HP + qy0], axis=-1)
    wts = jnp.stack([wx0 * wy0, wx1 * wy1, wx0 * wy1, wx1 * wy0], axis=-1)
    idx = jnp.transpose(idx, (1, 2, 0, 3)).reshape(HW, TAPS4)
    wts = jnp.transpose(wts, (1, 2, 0, 3)).reshape(HW, TAPS4)
    return idx, wts


def kernel(x, W_off, b_off, W_d, b_d):
    offset = lax.conv_general_dilated(
        x, W_off, window_strides=(1, 1), padding="VALID",
        dimension_numbers=("NCHW", "OIHW", "NCHW"))[0] + b_off[:, None, None]

    idx, wts = _indices_and_weights(offset)
    idx = jnp.pad(idx, ((0, NPIX - HW), (0, 0))).reshape(-1)
    wts = jnp.pad(wts, ((0, NPIX - HW), (0, 0))).reshape(-1)

    x_pad = jnp.pad(x[0], ((0, 0), (PAD, PAD), (PAD, PAD)))
    x_padT = x_pad.reshape(C, NROWS).T
    Wn = jnp.transpose(W_d.reshape(C, C, NTAP), (2, 1, 0))
    Wn = jnp.pad(Wn, ((0, 0), (0, 0), (0, 128 - C)))
    table = jnp.einsum("rc,ncd->nrd", x_padT, Wn).reshape(NTAP * NROWS, 128)

    out_rows = _sc_gather_combine(table, idx, wts)

    out = out_rows[:HW].T + b_d[:, None]
    return out.reshape(1, C, H, H)
